# Initial kernel scaffold; baseline (speedup 1.0000x reference)
import jax
import jax.numpy as jnp
from jax.experimental import pallas as pl


def kernel(x, gso, weight):
    raise NotImplementedError("write your pallas kernel here")


# speedup vs baseline: 1.0000x; 1.0000x over previous
import functools

import jax
import jax.numpy as jnp
from jax.experimental import pallas as pl
from jax.experimental.pallas import tpu as pltpu

BM = 400


def _dot(a, b):
    return jax.lax.dot_general(
        a, b, (((1,), (0,)), ((), ())), preferred_element_type=jnp.float32)


def _mm_body(gso_ref, x_ref, o_ref):
    o_ref[...] = _dot(gso_ref[...].astype(jnp.bfloat16),
                      x_ref[...].astype(jnp.bfloat16))


def _fused_body(gso_ref, t1k_ref, t1i_ref, xi_ref, w1_ref, w2_ref,
                w02_ref, o_ref):
    acc = _dot(gso_ref[...].astype(jnp.bfloat16),
               t1k_ref[...].astype(jnp.bfloat16))
    o_ref[...] = (_dot(2.0 * acc, w2_ref[...])
                  + _dot(t1i_ref[...], w1_ref[...])
                  + _dot(xi_ref[...], w02_ref[...]))


def kernel(x, gso, weight):
    n, in_size = x.shape
    out_size = weight.shape[2]
    nm = n // BM

    flops_mm = 2 * n * n * in_size
    mm = pl.pallas_call(
        _mm_body,
        grid=(nm,),
        in_specs=[
            pl.BlockSpec((BM, n), lambda i: (i, 0)),
            pl.BlockSpec((n, in_size), lambda i: (0, 0)),
        ],
        out_specs=pl.BlockSpec((BM, in_size), lambda i: (i, 0)),
        out_shape=jax.ShapeDtypeStruct((n, in_size), jnp.float32),
        compiler_params=pltpu.CompilerParams(
            dimension_semantics=("arbitrary",),
        ),
        cost_estimate=pl.CostEstimate(
            flops=flops_mm, bytes_accessed=gso.size * 4, transcendentals=0),
    )
    t1 = mm(gso, x)

    w0, w1, w2 = weight[0], weight[1], weight[2]
    w02 = w0 - w2

    wspec = pl.BlockSpec((in_size, out_size), lambda i: (0, 0))
    fused = pl.pallas_call(
        _fused_body,
        grid=(nm,),
        in_specs=[
            pl.BlockSpec((BM, n), lambda i: (i, 0)),
            pl.BlockSpec((n, in_size), lambda i: (0, 0)),
            pl.BlockSpec((BM, in_size), lambda i: (i, 0)),
            pl.BlockSpec((BM, in_size), lambda i: (i, 0)),
            wspec, wspec, wspec,
        ],
        out_specs=pl.BlockSpec((BM, out_size), lambda i: (i, 0)),
        out_shape=jax.ShapeDtypeStruct((n, out_size), jnp.float32),
        compiler_params=pltpu.CompilerParams(
            dimension_semantics=("arbitrary",),
        ),
        cost_estimate=pl.CostEstimate(
            flops=flops_mm, bytes_accessed=gso.size * 4, transcendentals=0),
    )
    return fused(gso, t1, t1, x, w1, w2, w02)


# speedup vs baseline: 1.0412x; 1.0412x over previous
import jax
import jax.numpy as jnp
from jax.experimental import pallas as pl
from jax.experimental.pallas import tpu as pltpu

BM = 400


def _dot(a, b):
    return jax.lax.dot_general(
        a, b, (((1,), (0,)), ((), ())), preferred_element_type=jnp.float32)


def _body(gso_ref, x_ref, w1_ref, w2_ref, w02_ref, o_ref, t1_ref):
    p = pl.program_id(0)
    i = pl.program_id(1)
    g = gso_ref[...].astype(jnp.bfloat16)

    @pl.when(p == 0)
    def _phase0():
        t1 = _dot(g, x_ref[...].astype(jnp.bfloat16))
        t1_ref[pl.ds(i * BM, BM), :] = t1.astype(jnp.bfloat16)

    @pl.when(p == 1)
    def _phase1():
        acc = _dot(g, t1_ref[...])
        t1i = t1_ref[pl.ds(i * BM, BM), :].astype(jnp.float32)
        xi = x_ref[pl.ds(i * BM, BM), :]
        o_ref[...] = (_dot(2.0 * acc, w2_ref[...])
                      + _dot(t1i, w1_ref[...])
                      + _dot(xi, w02_ref[...]))


def kernel(x, gso, weight):
    n, in_size = x.shape
    out_size = weight.shape[2]
    nm = n // BM

    w0, w1, w2 = weight[0], weight[1], weight[2]
    w02 = w0 - w2

    full = pl.BlockSpec((n, in_size), lambda p, i: (0, 0))
    wspec = pl.BlockSpec((in_size, out_size), lambda p, i: (0, 0))
    fused = pl.pallas_call(
        _body,
        grid=(2, nm),
        in_specs=[
            pl.BlockSpec((BM, n), lambda p, i: (i, 0)),
            full, wspec, wspec, wspec,
        ],
        out_specs=pl.BlockSpec((BM, out_size), lambda p, i: (i * p, 0)),
        out_shape=jax.ShapeDtypeStruct((n, out_size), jnp.float32),
        scratch_shapes=[pltpu.VMEM((n, in_size), jnp.bfloat16)],
        compiler_params=pltpu.CompilerParams(
            dimension_semantics=("arbitrary", "arbitrary"),
        ),
        cost_estimate=pl.CostEstimate(
            flops=4 * n * n * in_size, bytes_accessed=2 * gso.size * 4,
            transcendentals=0),
    )
    return fused(gso, x, w1, w2, w02)
